# baseline (speedup 1.0000x reference)
import functools

import jax
import jax.numpy as jnp
from jax import lax
from jax.experimental import pallas as pl
from jax.experimental.pallas import tpu as pltpu
from jax.experimental.pallas import tpu_sc as plsc

N = 10000
E = 320000
D = 128

NC = 2
NS = 16
NW = NC * NS
C = 64
NP = 4
KPP = 40
KPT = NP * KPP
EPT = KPT * C
EPAD = NW * EPT
NACC = 10112
RPT = NACC // NS


NB = 5


def _sc_agg_body(W, do_deg, feat, srcr, dstr, zeros, ones_in, *rest):
    if do_deg:
        acc_out, deg_out, src_v, dst_v, bufs, gsems, ssems, acc_sh = rest
    else:
        acc_out, src_v, dst_v, bufs, gsems, ssems, acc_sh = rest
    c = lax.axis_index("c")
    s = lax.axis_index("s")

    if do_deg:
        pltpu.sync_copy(zeros.at[pl.ds(s * RPT, RPT)],
                        acc_sh.at[pl.ds(s * RPT, RPT)])
        pltpu.sync_copy(ones_in, bufs[0])
        plsc.subcore_barrier()

        def fire(j, carry):
            pltpu.async_copy(bufs[0], acc_sh.at[dst_v.at[j]], ssems[0],
                             add=True)
            return carry

        def drain(j, carry):
            pltpu.make_async_copy(bufs[0], acc_sh.at[dst_v.at[0]],
                                  ssems[0]).wait()
            return carry

        for ip in range(NP):
            pltpu.sync_copy(dstr.at[c, s, ip], dst_v)
            lax.fori_loop(0, KPP, fire, 0)
            lax.fori_loop(0, KPP, drain, 0)
        plsc.subcore_barrier()
        pltpu.sync_copy(acc_sh.at[pl.ds(s * RPT, RPT)],
                        deg_out.at[c, pl.ds(s * RPT, RPT)])

    def start_gather(j, b):
        pltpu.async_copy(feat.at[src_v.at[pl.ds(j * C, C)]], bufs[b],
                         gsems[b])

    def wait_gather(j, b):
        pltpu.make_async_copy(feat.at[src_v.at[pl.ds(j * C, C)]], bufs[b],
                              gsems[b]).wait()

    def start_scatter(j, b):
        pltpu.async_copy(bufs[b], acc_sh.at[dst_v.at[j]], ssems[b], add=True)

    def wait_scatter(b):
        pltpu.make_async_copy(bufs[b], acc_sh.at[dst_v.at[0]],
                              ssems[b]).wait()

    for ip in range(NP):
        base = ((c * NS + s) * NP + ip) * (KPP * C)
        pltpu.sync_copy(srcr.at[pl.ds(base, KPP * C)], src_v)
        pltpu.sync_copy(dstr.at[c, s, ip], dst_v)

        start_gather(0, 0)
        start_gather(1, 1)
        start_gather(2, 2)
        start_gather(3, 3)
        if ip == 0:
            pltpu.sync_copy(zeros.at[pl.ds(s * RPT, RPT)],
                            acc_sh.at[pl.ds(s * RPT, RPT)])
            plsc.subcore_barrier()

        def quad(p, carry):
            for off in range(NB):
                j = NB * p + off
                wait_gather(j, off)
                start_scatter(j, off)
                nxt = (off + 4) % NB

                @pl.when(j >= 1)
                def _():
                    wait_scatter(nxt)

                @pl.when(j + 4 < KPP)
                def _():
                    start_gather(j + 4, nxt)
            return carry

        lax.fori_loop(0, KPP // NB, quad, 0)
        wait_scatter((KPP - 1) % NB)
    plsc.subcore_barrier()

    pltpu.sync_copy(acc_sh.at[pl.ds(s * RPT, RPT)],
                    acc_out.at[c, pl.ds(s * RPT, RPT)])


def _make_sc_agg(W, do_deg):
    mesh = plsc.VectorSubcoreMesh(core_axis_name="c", subcore_axis_name="s")
    outs = [jax.ShapeDtypeStruct((NC, NACC, W), jnp.float32)]
    if do_deg:
        outs = outs + [jax.ShapeDtypeStruct((NC, NACC, W), jnp.float32)]
    return pl.kernel(
        functools.partial(_sc_agg_body, W, do_deg),
        out_type=outs,
        mesh=mesh,
        scratch_types=[
            pltpu.VMEM((KPP * C,), jnp.int32),
            pltpu.VMEM((KPP, C), jnp.int32),
            [pltpu.VMEM((C, W), jnp.float32) for _ in range(NB)],
            [pltpu.SemaphoreType.DMA for _ in range(NB)],
            [pltpu.SemaphoreType.DMA for _ in range(NB)],
            pltpu.VMEM_SHARED((NACC, W), jnp.float32),
        ],
    )


def _right_body(xr, wr, b, out):
    out[...] = lax.dot_general(xr[...], wr[...], (((1,), (1,)), ((), ())),
                               preferred_element_type=jnp.float32) + b[...]


def _combine1_body(parts, degp, rterm, wl, h_out, invd_out):
    agg = parts[0] + parts[1]
    deg = degp[0, :, 0] + degp[1, :, 0]
    invd = 1.0 / jnp.maximum(deg, 1.0)
    mean = agg * invd[:, None]
    o = lax.dot_general(mean, wl[...], (((1,), (1,)), ((), ())),
                        preferred_element_type=jnp.float32)
    o = o + rterm[...]
    h_out[...] = jnp.maximum(o, 0.0)
    invd_out[...] = invd[:, None]


def _combine2_body(parts, invd, rterm, wl, out):
    agg = parts[0] + parts[1]
    mean = agg * invd[...]
    o = lax.dot_general(mean, wl[...], (((1,), (1,)), ((), ())),
                        preferred_element_type=jnp.float32)
    o = o + rterm[...]
    m = jnp.max(o, axis=1, keepdims=True)
    sh = o - m
    lse = jnp.log(jnp.sum(jnp.exp(sh), axis=1, keepdims=True))
    out[...] = sh - lse


_R = 1024
_GRID = (N + _R - 1) // _R


def _right(xr, wr, b):
    return pl.pallas_call(
        _right_body,
        grid=(_GRID,),
        in_specs=[
            pl.BlockSpec((_R, D), lambda i: (i, 0)),
            pl.BlockSpec((D, D), lambda i: (0, 0)),
            pl.BlockSpec((1, D), lambda i: (0, 0)),
        ],
        out_specs=pl.BlockSpec((_R, D), lambda i: (i, 0)),
        out_shape=jax.ShapeDtypeStruct((N, D), jnp.float32),
    )(xr, wr, b)


def _combine1(parts, degp, rterm, wl):
    return pl.pallas_call(
        _combine1_body,
        grid=(_GRID,),
        in_specs=[
            pl.BlockSpec((NC, _R, D), lambda i: (0, i, 0)),
            pl.BlockSpec((NC, _R, D), lambda i: (0, i, 0)),
            pl.BlockSpec((_R, D), lambda i: (i, 0)),
            pl.BlockSpec((D, D), lambda i: (0, 0)),
        ],
        out_specs=[
            pl.BlockSpec((_R, D), lambda i: (i, 0)),
            pl.BlockSpec((_R, 1), lambda i: (i, 0)),
        ],
        out_shape=[
            jax.ShapeDtypeStruct((N, D), jnp.float32),
            jax.ShapeDtypeStruct((N, 1), jnp.float32),
        ],
    )(parts, degp, rterm, wl)


def _combine2(parts, invd, rterm, wl):
    return pl.pallas_call(
        _combine2_body,
        grid=(_GRID,),
        in_specs=[
            pl.BlockSpec((NC, _R, D), lambda i: (0, i, 0)),
            pl.BlockSpec((_R, 1), lambda i: (i, 0)),
            pl.BlockSpec((_R, D), lambda i: (i, 0)),
            pl.BlockSpec((D, D), lambda i: (0, 0)),
        ],
        out_specs=pl.BlockSpec((_R, D), lambda i: (i, 0)),
        out_shape=jax.ShapeDtypeStruct((N, D), jnp.float32),
    )(parts, invd, rterm, wl)


_kernel_cache = {}


def _get(name):
    if name not in _kernel_cache:
        _kernel_cache[name] = _make_sc_agg(D, do_deg=(name == "aggdeg"))
    return _kernel_cache[name]


def kernel(x, edge_index, W1_l, W1_r, b1, W2_l, W2_r, b2):
    src = edge_index[0]
    dst = edge_index[1]
    pad = EPAD - E
    pad_idx = jnp.arange(pad, dtype=jnp.int32)
    src_r = jnp.concatenate([src, pad_idx % N])
    dst_r = jnp.concatenate(
        [dst, N + pad_idx % (NACC - N)]).reshape(NC, NS, NP, KPP, C)
    zeros = jnp.zeros((NACC, D), jnp.float32)
    ones_in = jnp.ones((C, D), jnp.float32)

    acc1, degp = _get("aggdeg")(x, src_r, dst_r, zeros, ones_in)
    r1 = _right(x, W1_r, b1.reshape(1, D))
    h, invd = _combine1(acc1, degp, r1, W1_l)
    (acc2,) = _get("agg")(h, src_r, dst_r, zeros, ones_in)
    r2 = _right(h, W2_r, b2.reshape(1, D))
    return _combine2(acc2, invd, r2, W2_l)


# speedup vs baseline: 1.0056x; 1.0056x over previous
import functools

import jax
import jax.numpy as jnp
from jax import lax
from jax.experimental import pallas as pl
from jax.experimental.pallas import tpu as pltpu
from jax.experimental.pallas import tpu_sc as plsc

N = 10000
E = 320000
D = 128

NC = 2
NS = 16
NW = NC * NS
C = 64
NP = 4
KPP = 40
KPT = NP * KPP
EPT = KPT * C
EPAD = NW * EPT
NACC = 10112
RPT = NACC // NS


NB = 5


def _sc_agg_body(W, do_deg, feat, srcr, dstr, zeros, ones_in, *rest):
    if do_deg:
        acc_out, deg_out, src_v, dst_v, bufs, gsems, ssems, acc_sh = rest
    else:
        acc_out, src_v, dst_v, bufs, gsems, ssems, acc_sh = rest
    c = lax.axis_index("c")
    s = lax.axis_index("s")

    if do_deg:
        pltpu.sync_copy(zeros.at[pl.ds(s * RPT, RPT)],
                        acc_sh.at[pl.ds(s * RPT, RPT)])
        pltpu.sync_copy(ones_in, bufs[0])
        plsc.subcore_barrier()

        def fire(j, carry):
            pltpu.async_copy(bufs[0], acc_sh.at[dst_v.at[j]], ssems[0],
                             add=True)
            return carry

        def drain(j, carry):
            pltpu.make_async_copy(bufs[0], acc_sh.at[dst_v.at[0]],
                                  ssems[0]).wait()
            return carry

        for ip in range(NP):
            pltpu.sync_copy(dstr.at[c, s, ip], dst_v)
            lax.fori_loop(0, KPP, fire, 0)
            lax.fori_loop(0, KPP, drain, 0)
        plsc.subcore_barrier()
        pltpu.sync_copy(acc_sh.at[pl.ds(s * RPT, RPT)],
                        deg_out.at[c, pl.ds(s * RPT, RPT)])

    def start_gather(j, b):
        pltpu.async_copy(feat.at[src_v.at[pl.ds(j * C, C)]], bufs[b],
                         gsems[b])

    def wait_gather(j, b):
        pltpu.make_async_copy(feat.at[src_v.at[pl.ds(j * C, C)]], bufs[b],
                              gsems[b]).wait()

    def start_scatter(j, b):
        pltpu.async_copy(bufs[b], acc_sh.at[dst_v.at[j]], ssems[b], add=True)

    def wait_scatter(b):
        pltpu.make_async_copy(bufs[b], acc_sh.at[dst_v.at[0]],
                              ssems[b]).wait()

    for ip in range(NP):
        base = ((c * NS + s) * NP + ip) * (KPP * C)
        pltpu.sync_copy(srcr.at[pl.ds(base, KPP * C)], src_v)
        pltpu.sync_copy(dstr.at[c, s, ip], dst_v)

        start_gather(0, 0)
        start_gather(1, 1)
        start_gather(2, 2)
        start_gather(3, 3)
        if ip == 0:
            pltpu.sync_copy(zeros.at[pl.ds(s * RPT, RPT)],
                            acc_sh.at[pl.ds(s * RPT, RPT)])
            plsc.subcore_barrier()

        def quad(p, carry):
            for off in range(NB):
                j = NB * p + off
                wait_gather(j, off)
                start_scatter(j, off)
                nxt = (off + 4) % NB

                @pl.when(j >= 1)
                def _():
                    wait_scatter(nxt)

                @pl.when(j + 4 < KPP)
                def _():
                    start_gather(j + 4, nxt)
            return carry

        lax.fori_loop(0, KPP // NB, quad, 0)
        wait_scatter((KPP - 1) % NB)
    plsc.subcore_barrier()

    pltpu.sync_copy(acc_sh.at[pl.ds(s * RPT, RPT)],
                    acc_out.at[c, pl.ds(s * RPT, RPT)])


def _make_sc_agg(W, do_deg):
    mesh = plsc.VectorSubcoreMesh(core_axis_name="c", subcore_axis_name="s")
    outs = [jax.ShapeDtypeStruct((NC, NACC, W), jnp.float32)]
    if do_deg:
        outs = outs + [jax.ShapeDtypeStruct((NC, NACC, W), jnp.float32)]
    return pl.kernel(
        functools.partial(_sc_agg_body, W, do_deg),
        out_type=outs,
        mesh=mesh,
        scratch_types=[
            pltpu.VMEM((KPP * C,), jnp.int32),
            pltpu.VMEM((KPP, C), jnp.int32),
            [pltpu.VMEM((C, W), jnp.float32) for _ in range(NB)],
            [pltpu.SemaphoreType.DMA for _ in range(NB)],
            [pltpu.SemaphoreType.DMA for _ in range(NB)],
            pltpu.VMEM_SHARED((NACC, W), jnp.float32),
        ],
    )


def _combine1_body(parts, degp, xr, wl, wr, b, h_out, invd_out):
    agg = parts[0] + parts[1]
    deg = degp[0, :, 0] + degp[1, :, 0]
    invd = 1.0 / jnp.maximum(deg, 1.0)
    mean = agg * invd[:, None]
    o = lax.dot_general(mean, wl[...], (((1,), (1,)), ((), ())),
                        preferred_element_type=jnp.float32)
    o = o + lax.dot_general(xr[...], wr[...], (((1,), (1,)), ((), ())),
                            preferred_element_type=jnp.float32)
    o = o + b[...]
    h_out[...] = jnp.maximum(o, 0.0)
    invd_out[...] = invd[:, None]


def _combine2_body(parts, invd, hr, wl, wr, b, out):
    agg = parts[0] + parts[1]
    mean = agg * invd[...]
    o = lax.dot_general(mean, wl[...], (((1,), (1,)), ((), ())),
                        preferred_element_type=jnp.float32)
    o = o + lax.dot_general(hr[...], wr[...], (((1,), (1,)), ((), ())),
                            preferred_element_type=jnp.float32)
    o = o + b[...]
    m = jnp.max(o, axis=1, keepdims=True)
    sh = o - m
    lse = jnp.log(jnp.sum(jnp.exp(sh), axis=1, keepdims=True))
    out[...] = sh - lse


_R = 1024
_GRID = (N + _R - 1) // _R


def _combine1(parts, degp, x, wl, wr, b):
    return pl.pallas_call(
        _combine1_body,
        grid=(_GRID,),
        in_specs=[
            pl.BlockSpec((NC, _R, D), lambda i: (0, i, 0)),
            pl.BlockSpec((NC, _R, D), lambda i: (0, i, 0)),
            pl.BlockSpec((_R, D), lambda i: (i, 0)),
            pl.BlockSpec((D, D), lambda i: (0, 0)),
            pl.BlockSpec((D, D), lambda i: (0, 0)),
            pl.BlockSpec((1, D), lambda i: (0, 0)),
        ],
        out_specs=[
            pl.BlockSpec((_R, D), lambda i: (i, 0)),
            pl.BlockSpec((_R, 1), lambda i: (i, 0)),
        ],
        out_shape=[
            jax.ShapeDtypeStruct((N, D), jnp.float32),
            jax.ShapeDtypeStruct((N, 1), jnp.float32),
        ],
    )(parts, degp, x, wl, wr, b)


def _combine2(parts, invd, h, wl, wr, b):
    return pl.pallas_call(
        _combine2_body,
        grid=(_GRID,),
        in_specs=[
            pl.BlockSpec((NC, _R, D), lambda i: (0, i, 0)),
            pl.BlockSpec((_R, 1), lambda i: (i, 0)),
            pl.BlockSpec((_R, D), lambda i: (i, 0)),
            pl.BlockSpec((D, D), lambda i: (0, 0)),
            pl.BlockSpec((D, D), lambda i: (0, 0)),
            pl.BlockSpec((1, D), lambda i: (0, 0)),
        ],
        out_specs=pl.BlockSpec((_R, D), lambda i: (i, 0)),
        out_shape=jax.ShapeDtypeStruct((N, D), jnp.float32),
    )(parts, invd, h, wl, wr, b)


_kernel_cache = {}


def _get(name):
    if name not in _kernel_cache:
        _kernel_cache[name] = _make_sc_agg(D, do_deg=(name == "aggdeg"))
    return _kernel_cache[name]


def kernel(x, edge_index, W1_l, W1_r, b1, W2_l, W2_r, b2):
    src = edge_index[0]
    dst = edge_index[1]
    pad = EPAD - E
    pad_idx = jnp.arange(pad, dtype=jnp.int32)
    src_r = jnp.concatenate([src, pad_idx % N])
    dst_r = jnp.concatenate(
        [dst, N + pad_idx % (NACC - N)]).reshape(NC, NS, NP, KPP, C)
    zeros = jnp.zeros((NACC, D), jnp.float32)
    ones_in = jnp.ones((C, D), jnp.float32)

    acc1, degp = _get("aggdeg")(x, src_r, dst_r, zeros, ones_in)
    h, invd = _combine1(acc1, degp, x, W1_l, W1_r, b1.reshape(1, D))
    (acc2,) = _get("agg")(h, src_r, dst_r, zeros, ones_in)
    return _combine2(acc2, invd, h, W2_l, W2_r, b2.reshape(1, D))
